# Initial kernel scaffold; baseline (speedup 1.0000x reference)
import jax
import jax.numpy as jnp
from jax.experimental import pallas as pl


def kernel(input_ids, wte, wpe):
    raise NotImplementedError("write your pallas kernel here")


# speedup vs baseline: 1.1275x; 1.1275x over previous
import functools

import jax
import jax.numpy as jnp
from jax import lax
from jax.experimental import pallas as pl
from jax.experimental.pallas import tpu as pltpu
from jax.experimental.pallas import tpu_sc as plsc

BATCH = 4
SEQ = 2048
D = 768
NC = 2
NS = 16
NW = NC * NS
PW = SEQ // NW
LANES = 16
NJ = D // LANES


def _emb_body(ids_hbm, wte_hbm, wpe_hbm, out_hbm, idx_v, wpe_v, rows_v):
    c = lax.axis_index("c")
    s = lax.axis_index("s")
    w = s * NC + c
    pbase = w * PW
    pltpu.sync_copy(wpe_hbm.at[pl.ds(pbase, PW)], wpe_v)
    for b in range(BATCH):
        pltpu.sync_copy(ids_hbm.at[pl.ds(b * SEQ + pbase, PW)], idx_v.at[b])
    for b in range(BATCH):
        pltpu.sync_copy(wte_hbm.at[idx_v.at[b]], rows_v)

        @pl.loop(0, PW)
        def _(r):
            for j in range(NJ):
                sl = pl.ds(j * LANES, LANES)
                plsc.addupdate(rows_v.at[r, sl], wpe_v[r, sl])

        pltpu.sync_copy(rows_v, out_hbm.at[pl.ds(b * SEQ + pbase, PW)])


def kernel(input_ids, wte, wpe):
    ids_flat = input_ids.reshape(-1).astype(jnp.int32)
    mesh = plsc.VectorSubcoreMesh(core_axis_name="c", subcore_axis_name="s")
    run = pl.kernel(
        _emb_body,
        out_type=jax.ShapeDtypeStruct((BATCH * SEQ, D), jnp.float32),
        mesh=mesh,
        scratch_types=[
            pltpu.VMEM((BATCH, PW), jnp.int32),
            pltpu.VMEM((PW, D), jnp.float32),
            pltpu.VMEM((PW, D), jnp.float32),
        ],
    )
    out = run(ids_flat, wte, wpe)
    return out.reshape(BATCH, SEQ, D)


# speedup vs baseline: 1.1519x; 1.0216x over previous
import jax
import jax.numpy as jnp
from jax import lax
from jax.experimental import pallas as pl
from jax.experimental.pallas import tpu as pltpu
from jax.experimental.pallas import tpu_sc as plsc

BATCH = 4
SEQ = 2048
D = 768
NC = 2
NS = 16
NW = NC * NS
PW = SEQ // NW
CH = 32
NCHUNK = BATCH * PW // CH
LANES = 16
NJ = D // LANES
NBUF = 3


def _emb_body(ids_hbm, wte_hbm, wpe_hbm, out_hbm,
              idx_v, wpe_v, buf0, buf1, buf2,
              wsem, g0, g1, g2, s0, s1, s2):
    c = lax.axis_index("c")
    s = lax.axis_index("s")
    w = s * NC + c
    pbase = w * PW
    bufs = (buf0, buf1, buf2)
    gsems = (g0, g1, g2)
    ssems = (s0, s1, s2)

    def gather(k):
        b, h = divmod(k, 2)
        idx = idx_v.at[b, pl.ds(h * CH, CH)]
        return pltpu.make_async_copy(wte_hbm.at[idx], bufs[k % NBUF],
                                     gsems[k % NBUF])

    def store(k):
        b, h = divmod(k, 2)
        row0 = b * SEQ + pbase + h * CH
        return pltpu.make_async_copy(bufs[k % NBUF],
                                     out_hbm.at[pl.ds(row0, CH)],
                                     ssems[k % NBUF])

    for b in range(BATCH):
        pltpu.sync_copy(ids_hbm.at[pl.ds(b * SEQ + pbase, PW)], idx_v.at[b])
    wpe_cp = pltpu.make_async_copy(wpe_hbm.at[pl.ds(pbase, PW)], wpe_v, wsem)
    wpe_cp.start()
    gather(0).start()
    gather(1).start()
    wpe_cp.wait()

    for k in range(NCHUNK):
        h = k % 2
        gather(k).wait()
        buf = bufs[k % NBUF]

        @pl.loop(0, CH)
        def _(r):
            for j in range(NJ):
                sl = pl.ds(j * LANES, LANES)
                plsc.addupdate(buf.at[r, sl], wpe_v[h * CH + r, sl])

        store(k).start()
        if k + 2 < NCHUNK:
            if k >= 1:
                store(k - 1).wait()
            gather(k + 2).start()

    for k in range(NCHUNK - 3, NCHUNK):
        store(k).wait()


def kernel(input_ids, wte, wpe):
    ids_flat = input_ids.reshape(-1).astype(jnp.int32)
    mesh = plsc.VectorSubcoreMesh(core_axis_name="c", subcore_axis_name="s")
    run = pl.kernel(
        _emb_body,
        out_type=jax.ShapeDtypeStruct((BATCH * SEQ, D), jnp.float32),
        mesh=mesh,
        scratch_types=[
            pltpu.VMEM((BATCH, PW), jnp.int32),
            pltpu.VMEM((PW, D), jnp.float32),
            pltpu.VMEM((CH, D), jnp.float32),
            pltpu.VMEM((CH, D), jnp.float32),
            pltpu.VMEM((CH, D), jnp.float32),
            pltpu.SemaphoreType.DMA,
            pltpu.SemaphoreType.DMA,
            pltpu.SemaphoreType.DMA,
            pltpu.SemaphoreType.DMA,
            pltpu.SemaphoreType.DMA,
            pltpu.SemaphoreType.DMA,
            pltpu.SemaphoreType.DMA,
        ],
    )
    out = run(ids_flat, wte, wpe)
    return out.reshape(BATCH, SEQ, D)
